# baseline (speedup 1.0000x reference)
import functools

import jax
import jax.numpy as jnp
import numpy as np
from jax import lax
from jax.experimental import pallas as pl
from jax.experimental.pallas import tpu as pltpu
from jax.experimental.pallas import tpu_sc as plsc

_MASK = np.array([
    [0, 1, 2, 3], [4, 5, 6, 7], [8, 9, 10, 11], [12, 13, 14, 15],
    [16, 17, 18, 19], [20, 21, 22, 23], [24, 25, 26, 27], [28, 29, 30, 31],
    [31, 30, 29, 28], [27, 26, 25, 24], [23, 22, 21, 20], [19, 18, 17, 16],
    [15, 14, 13, 12], [11, 10, 9, 8], [7, 6, 5, 4], [3, 2, 1, 0],
    [0, 8, 16, 24], [1, 9, 17, 25], [2, 10, 18, 26], [3, 11, 19, 27],
    [4, 12, 20, 28], [5, 13, 21, 29], [6, 14, 22, 30], [7, 15, 23, 31],
    [7, 15, 23, 31], [6, 14, 22, 30], [5, 13, 21, 29], [4, 12, 20, 28],
    [3, 11, 19, 27], [2, 10, 18, 26], [1, 9, 17, 25], [0, 8, 16, 24],
], dtype=np.int32)

_R, _D = _MASK.shape
_C = 64
_RC = _R * _C
_F = int(_MASK.max()) + 1
_DC = _D * _C
_FC = 2 * _F * _C
_HALF_LOG_2PI = float(0.5 * np.log(2.0 * np.pi))
_SIDX = (_MASK[:, :, None] * _C
         + np.arange(_C, dtype=np.int32)[None, None, :]).reshape(_R, _DC)

_info = plsc.get_sparse_core_info()
_NC, _NS, _L = _info.num_cores, _info.num_subcores, _info.num_lanes


def _sc_build_w(sidx, loc3, scale3):
    mesh = plsc.VectorSubcoreMesh(core_axis_name="c", subcore_axis_name="s")

    @functools.partial(
        pl.kernel, mesh=mesh,
        out_type=jax.ShapeDtypeStruct((_R, _FC), jnp.float32),
        scratch_types=[
            pltpu.VMEM((_DC,), jnp.int32),
            pltpu.VMEM((_DC,), jnp.float32),
            pltpu.VMEM((_DC,), jnp.float32),
            pltpu.VMEM((_FC,), jnp.float32),
        ],
    )
    def k(sidx_hbm, loc_hbm, scale_hbm, w_hbm, idx_v, loc_v, sc_v, w_v):
        wid = lax.axis_index("s") * _NC + lax.axis_index("c")
        pltpu.sync_copy(sidx_hbm.at[wid], idx_v)
        pltpu.sync_copy(loc_hbm.at[wid], loc_v)
        pltpu.sync_copy(scale_hbm.at[wid], sc_v)
        for j in range(_DC // _L):
            sl = pl.ds(j * _L, _L)
            l = loc_v[sl]
            s = sc_v[sl]
            var = s * s
            loc_v[sl] = l / var
            sc_v[sl] = -0.5 / var
        iota = lax.iota(jnp.int32, _L)
        zero = jnp.zeros((_L,), jnp.float32)
        for f in range(_F):
            for cc in range(_C // _L):
                pos = iota + (f * _C + cc * _L)
                acc_l = zero
                acc_q = zero
                for d in range(_D):
                    sl = pl.ds(d * _C + cc * _L, _L)
                    hit = idx_v[sl] == pos
                    acc_l = acc_l + jnp.where(hit, loc_v[sl], zero)
                    acc_q = acc_q + jnp.where(hit, sc_v[sl], zero)
                w_v[pl.ds(f * _C + cc * _L, _L)] = acc_l
                w_v[pl.ds(_F * _C + f * _C + cc * _L, _L)] = acc_q
        pltpu.sync_copy(w_v, w_hbm.at[wid])

    return k(sidx, loc3, scale3)


def _tc_body(w_ref, locd_ref, scaled_ref, x_ref, out_ref):
    locd = locd_ref[...]
    scaled = scaled_ref[...]
    cterm = (-0.5 / (scaled * scaled)) * locd * locd \
        - jnp.log(scaled) - _HALF_LOG_2PI
    bias = jnp.sum(cterm, axis=0, keepdims=True)
    xb = x_ref[...]
    z = jnp.concatenate([xb, xb * xb], axis=1)
    acc = jax.lax.dot_general(
        z.astype(jnp.bfloat16), w_ref[...].astype(jnp.bfloat16),
        (((1,), (0,)), ((), ())),
        preferred_element_type=jnp.float32)
    out_ref[...] = acc + bias


def kernel(x, loc, scale):
    batch = x.shape[0]
    bb = 1024
    locd = loc.transpose(2, 0, 1).reshape(_D, _RC)
    scaled = scale.transpose(2, 0, 1).reshape(_D, _RC)
    loc3 = loc.transpose(0, 2, 1).reshape(_R, _DC)
    scale3 = scale.transpose(0, 2, 1).reshape(_R, _DC)
    w3 = _sc_build_w(jnp.asarray(_SIDX), loc3, scale3)
    w = w3.reshape(_R, 2 * _F, _C).transpose(1, 0, 2).reshape(2 * _F, _RC)
    grid = (batch // bb,)
    out = pl.pallas_call(
        _tc_body,
        grid=grid,
        in_specs=[
            pl.BlockSpec((2 * _F, _RC), lambda i: (0, 0)),
            pl.BlockSpec((_D, _RC), lambda i: (0, 0)),
            pl.BlockSpec((_D, _RC), lambda i: (0, 0)),
            pl.BlockSpec((bb, _F), lambda i: (i, 0)),
        ],
        out_specs=pl.BlockSpec((bb, _RC), lambda i: (i, 0)),
        out_shape=jax.ShapeDtypeStruct((batch, _RC), jnp.float32),
        compiler_params=pltpu.CompilerParams(
            dimension_semantics=("parallel",)),
    )(w, locd, scaled, x)
    return out.reshape(batch, _R, _C)


# speedup vs baseline: 1.2410x; 1.2410x over previous
import jax
import jax.numpy as jnp
import numpy as np
from jax.experimental import pallas as pl
from jax.experimental.pallas import tpu as pltpu

_MASK = np.array([
    [0, 1, 2, 3], [4, 5, 6, 7], [8, 9, 10, 11], [12, 13, 14, 15],
    [16, 17, 18, 19], [20, 21, 22, 23], [24, 25, 26, 27], [28, 29, 30, 31],
    [31, 30, 29, 28], [27, 26, 25, 24], [23, 22, 21, 20], [19, 18, 17, 16],
    [15, 14, 13, 12], [11, 10, 9, 8], [7, 6, 5, 4], [3, 2, 1, 0],
    [0, 8, 16, 24], [1, 9, 17, 25], [2, 10, 18, 26], [3, 11, 19, 27],
    [4, 12, 20, 28], [5, 13, 21, 29], [6, 14, 22, 30], [7, 15, 23, 31],
    [7, 15, 23, 31], [6, 14, 22, 30], [5, 13, 21, 29], [4, 12, 20, 28],
    [3, 11, 19, 27], [2, 10, 18, 26], [1, 9, 17, 25], [0, 8, 16, 24],
], dtype=np.int32)

_R, _D = _MASK.shape
_C = 64
_RC = _R * _C
_F = int(_MASK.max()) + 1
_HALF_LOG_2PI = float(0.5 * np.log(2.0 * np.pi))
_COLREG = np.repeat(_MASK.T, _C, axis=1)


def _body(colreg_ref, locd_ref, scaled_ref, x_ref, out_ref):
    locd = locd_ref[...]
    scaled = scaled_ref[...]
    colreg = colreg_ref[...]
    var = scaled * scaled
    quad = -0.5 / var
    lin = locd / var
    cterm = quad * locd * locd - jnp.log(scaled) - _HALF_LOG_2PI
    bias = jnp.sum(cterm, axis=0, keepdims=True)
    ncol = colreg.shape[1]
    iota = jax.lax.broadcasted_iota(jnp.int32, (_F, ncol), 0)
    w_lin = jnp.zeros((_F, ncol), jnp.float32)
    w_quad = jnp.zeros((_F, ncol), jnp.float32)
    for d in range(_D):
        m = colreg[d:d + 1, :] == iota
        w_lin = w_lin + jnp.where(m, lin[d:d + 1, :], 0.0)
        w_quad = w_quad + jnp.where(m, quad[d:d + 1, :], 0.0)
    w = jnp.concatenate([w_lin, w_quad], axis=0).astype(jnp.bfloat16)

    xb = x_ref[...]
    z = jnp.concatenate([xb, xb * xb], axis=1)
    acc = jax.lax.dot_general(
        z.astype(jnp.bfloat16), w, (((1,), (0,)), ((), ())),
        preferred_element_type=jnp.float32)
    out_ref[...] = acc + bias


def kernel(x, loc, scale):
    batch = x.shape[0]
    bb = 1024
    cb = _RC // 2
    locd = loc.transpose(2, 0, 1).reshape(_D, _RC)
    scaled = scale.transpose(2, 0, 1).reshape(_D, _RC)
    colreg = jnp.asarray(_COLREG)
    grid = (batch // bb, _RC // cb)
    out = pl.pallas_call(
        _body,
        grid=grid,
        in_specs=[
            pl.BlockSpec((_D, cb), lambda i, j: (0, j)),
            pl.BlockSpec((_D, cb), lambda i, j: (0, j)),
            pl.BlockSpec((_D, cb), lambda i, j: (0, j)),
            pl.BlockSpec((bb, _F), lambda i, j: (i, 0)),
        ],
        out_specs=pl.BlockSpec((bb, cb), lambda i, j: (i, j)),
        out_shape=jax.ShapeDtypeStruct((batch, _RC), jnp.float32),
        compiler_params=pltpu.CompilerParams(
            dimension_semantics=("parallel", "parallel")),
    )(colreg, locd, scaled, x)
    return out.reshape(batch, _R, _C)


# speedup vs baseline: 1.2867x; 1.0368x over previous
import jax
import jax.numpy as jnp
import numpy as np
from jax.experimental import pallas as pl
from jax.experimental.pallas import tpu as pltpu

_MASK = np.array([
    [0, 1, 2, 3], [4, 5, 6, 7], [8, 9, 10, 11], [12, 13, 14, 15],
    [16, 17, 18, 19], [20, 21, 22, 23], [24, 25, 26, 27], [28, 29, 30, 31],
    [31, 30, 29, 28], [27, 26, 25, 24], [23, 22, 21, 20], [19, 18, 17, 16],
    [15, 14, 13, 12], [11, 10, 9, 8], [7, 6, 5, 4], [3, 2, 1, 0],
    [0, 8, 16, 24], [1, 9, 17, 25], [2, 10, 18, 26], [3, 11, 19, 27],
    [4, 12, 20, 28], [5, 13, 21, 29], [6, 14, 22, 30], [7, 15, 23, 31],
    [7, 15, 23, 31], [6, 14, 22, 30], [5, 13, 21, 29], [4, 12, 20, 28],
    [3, 11, 19, 27], [2, 10, 18, 26], [1, 9, 17, 25], [0, 8, 16, 24],
], dtype=np.int32)

_R, _D = _MASK.shape
_C = 64
_RC = _R * _C
_F = int(_MASK.max()) + 1
_HALF_LOG_2PI = float(0.5 * np.log(2.0 * np.pi))
_COLREG = np.repeat(_MASK.T, _C, axis=1)


def _body(colreg_ref, locd_ref, scaled_ref, x_ref, out_ref):
    locd = locd_ref[...]
    scaled = scaled_ref[...]
    colreg = colreg_ref[...]
    var = scaled * scaled
    quad = -0.5 / var
    lin = locd / var
    cterm = quad * locd * locd - jnp.log(scaled) - _HALF_LOG_2PI
    bias = jnp.sum(cterm, axis=0, keepdims=True)
    iota = jax.lax.broadcasted_iota(jnp.int32, (_F, _RC), 0)
    w_lin = jnp.zeros((_F, _RC), jnp.float32)
    w_quad = jnp.zeros((_F, _RC), jnp.float32)
    for d in range(_D):
        m = colreg[d:d + 1, :] == iota
        w_lin = w_lin + jnp.where(m, lin[d:d + 1, :], 0.0)
        w_quad = w_quad + jnp.where(m, quad[d:d + 1, :], 0.0)
    w = jnp.concatenate([w_lin, w_quad], axis=0).astype(jnp.bfloat16)

    xb = x_ref[...]
    z = jnp.concatenate([xb, xb * xb], axis=1)
    acc = jax.lax.dot_general(
        z.astype(jnp.bfloat16), w, (((1,), (0,)), ((), ())),
        preferred_element_type=jnp.float32)
    out_ref[...] = acc + bias


def kernel(x, loc, scale):
    batch = x.shape[0]
    bb = 1024
    locd = loc.transpose(2, 0, 1).reshape(_D, _RC)
    scaled = scale.transpose(2, 0, 1).reshape(_D, _RC)
    colreg = jnp.asarray(_COLREG)
    grid = (batch // bb,)
    out = pl.pallas_call(
        _body,
        grid=grid,
        in_specs=[
            pl.BlockSpec((_D, _RC), lambda i: (0, 0)),
            pl.BlockSpec((_D, _RC), lambda i: (0, 0)),
            pl.BlockSpec((_D, _RC), lambda i: (0, 0)),
            pl.BlockSpec((bb, _F), lambda i: (i, 0)),
        ],
        out_specs=pl.BlockSpec((bb, _RC), lambda i: (i, 0)),
        out_shape=jax.ShapeDtypeStruct((batch, _RC), jnp.float32),
        compiler_params=pltpu.CompilerParams(
            dimension_semantics=("parallel",)),
    )(colreg, locd, scaled, x)
    return out.reshape(batch, _R, _C)


# speedup vs baseline: 1.2874x; 1.0005x over previous
import jax
import jax.numpy as jnp
import numpy as np
from jax.experimental import pallas as pl
from jax.experimental.pallas import tpu as pltpu

_MASK = np.array([
    [0, 1, 2, 3], [4, 5, 6, 7], [8, 9, 10, 11], [12, 13, 14, 15],
    [16, 17, 18, 19], [20, 21, 22, 23], [24, 25, 26, 27], [28, 29, 30, 31],
    [31, 30, 29, 28], [27, 26, 25, 24], [23, 22, 21, 20], [19, 18, 17, 16],
    [15, 14, 13, 12], [11, 10, 9, 8], [7, 6, 5, 4], [3, 2, 1, 0],
    [0, 8, 16, 24], [1, 9, 17, 25], [2, 10, 18, 26], [3, 11, 19, 27],
    [4, 12, 20, 28], [5, 13, 21, 29], [6, 14, 22, 30], [7, 15, 23, 31],
    [7, 15, 23, 31], [6, 14, 22, 30], [5, 13, 21, 29], [4, 12, 20, 28],
    [3, 11, 19, 27], [2, 10, 18, 26], [1, 9, 17, 25], [0, 8, 16, 24],
], dtype=np.int32)

_R, _D = _MASK.shape
_C = 64
_RC = _R * _C
_F = int(_MASK.max()) + 1
_HALF_LOG_2PI = float(0.5 * np.log(2.0 * np.pi))
_COLREG = np.repeat(_MASK.T, _C, axis=1)


def _body(colreg_ref, locd_ref, scaled_ref, x_ref, out_ref, w_ref, c_ref):
    @pl.when(pl.program_id(0) == 0)
    def _build_w():
        locd = locd_ref[...]
        scaled = scaled_ref[...]
        colreg = colreg_ref[...]
        var = scaled * scaled
        quad = -0.5 / var
        lin = locd / var
        cterm = quad * locd * locd - jnp.log(scaled) - _HALF_LOG_2PI
        c_ref[...] = jnp.sum(cterm, axis=0, keepdims=True)
        iota = jax.lax.broadcasted_iota(jnp.int32, (_F, _RC), 0)
        w_lin = jnp.zeros((_F, _RC), jnp.float32)
        w_quad = jnp.zeros((_F, _RC), jnp.float32)
        for d in range(_D):
            m = colreg[d:d + 1, :] == iota
            w_lin = w_lin + jnp.where(m, lin[d:d + 1, :], 0.0)
            w_quad = w_quad + jnp.where(m, quad[d:d + 1, :], 0.0)
        w_ref[0:_F, :] = w_lin.astype(jnp.bfloat16)
        w_ref[_F:2 * _F, :] = w_quad.astype(jnp.bfloat16)

    xb = x_ref[...]
    z = jnp.concatenate([xb, xb * xb], axis=1)
    acc = jax.lax.dot_general(
        z.astype(jnp.bfloat16), w_ref[...], (((1,), (0,)), ((), ())),
        preferred_element_type=jnp.float32)
    out_ref[...] = acc + c_ref[...]


def kernel(x, loc, scale):
    batch = x.shape[0]
    bb = 1024
    locd = loc.transpose(2, 0, 1).reshape(_D, _RC)
    scaled = scale.transpose(2, 0, 1).reshape(_D, _RC)
    colreg = jnp.asarray(_COLREG)
    grid = (batch // bb,)
    out = pl.pallas_call(
        _body,
        grid=grid,
        in_specs=[
            pl.BlockSpec((_D, _RC), lambda i: (0, 0)),
            pl.BlockSpec((_D, _RC), lambda i: (0, 0)),
            pl.BlockSpec((_D, _RC), lambda i: (0, 0)),
            pl.BlockSpec((bb, _F), lambda i: (i, 0)),
        ],
        out_specs=pl.BlockSpec((bb, _RC), lambda i: (i, 0)),
        out_shape=jax.ShapeDtypeStruct((batch, _RC), jnp.float32),
        scratch_shapes=[
            pltpu.VMEM((2 * _F, _RC), jnp.bfloat16),
            pltpu.VMEM((1, _RC), jnp.float32),
        ],
    )(colreg, locd, scaled, x)
    return out.reshape(batch, _R, _C)
